# baseline (speedup 1.0000x reference)
import functools

import jax
import jax.numpy as jnp
from jax import lax
from jax.experimental import pallas as pl
from jax.experimental.pallas import tpu as pltpu
from jax.experimental.pallas import tpu_sc as plsc

_DIM = 128
_B = 512
_MARGIN = 0.1
_MARGIN1 = 0.05
_INF = 4.0
_NW = 32
_BPW = _B // _NW
_NCE = 14
_NRE = 2
_ROWS = 21 * _BPW
_PROWS = 48
_L = 16


@functools.cache
def _get_sc_kernel():
    mesh = plsc.VectorSubcoreMesh(core_axis_name="c", subcore_axis_name="s")

    @functools.partial(
        pl.kernel,
        mesh=mesh,
        out_type=jax.ShapeDtypeStruct((_NW * _PROWS, 128), jnp.float32),
        scratch_types=[
            pltpu.VMEM((2, 112), jnp.int32),
            pltpu.VMEM((_NRE * _BPW,), jnp.int32),
            pltpu.VMEM((_NCE * _BPW, 2 * _DIM), jnp.float32),
            pltpu.VMEM((_NRE * _BPW, 2 * _DIM), jnp.float32),
            pltpu.VMEM((_PROWS, 128), jnp.float32),
            pltpu.SemaphoreType.DMA,
        ],
    )
    def _sc_loss(ce_hbm, re_hbm, ceidx_hbm, reidx_hbm, out_hbm,
                 ceidx_v, reidx_v, ce_rows, re_rows, buf, sem):
        w = lax.axis_index("s") * 2 + lax.axis_index("c")
        pltpu.sync_copy(ceidx_hbm.at[w], ceidx_v)
        cp0 = pltpu.async_copy(ce_hbm.at[ceidx_v.at[0]], ce_rows.at[pl.ds(0, 112)], sem)
        cp1 = pltpu.async_copy(ce_hbm.at[ceidx_v.at[1]], ce_rows.at[pl.ds(112, 112)], sem)
        pltpu.sync_copy(reidx_hbm.at[w], reidx_v)
        cp2 = pltpu.async_copy(re_hbm.at[reidx_v], re_rows, sem)

        m = jnp.float32(_MARGIN)
        m1 = jnp.float32(_MARGIN1)
        lanes = lax.iota(jnp.int32, _L)

        def group(slot, i, g):
            a = ce_rows[slot * _BPW + i, pl.ds(g * _L, _L)]
            b = jnp.abs(ce_rows[slot * _BPW + i, pl.ds(_DIM + g * _L, _L)])
            return a, b

        def put(row, acc_or_parts):
            if isinstance(acc_or_parts, list):
                acc = acc_or_parts[0] * acc_or_parts[0]
                for p in acc_or_parts[1:]:
                    acc = acc + p * p
            else:
                acc = acc_or_parts
            buf[row // 8, pl.ds((row % 8) * _L, _L)] = acc

        def head_2op(base, cslot, dslot, sgn_r, bias):
            def body(i, _):
                t1 = []
                t2 = []
                t3 = []
                for g in range(8):
                    c1, cr = group(cslot, i, g)
                    d1, dr = group(dslot, i, g)
                    euc = jnp.abs(c1 - d1)
                    if sgn_r == 0:
                        t = euc + cr - dr + bias
                    elif sgn_r == 1:
                        t = cr + dr + bias - euc
                    else:
                        t = euc - cr - dr + bias
                    t1.append(jnp.maximum(t, 0.0))
                    t2.append(jnp.maximum(m - cr, 0.0))
                    t3.append(jnp.maximum(m - dr, 0.0))
                put(base + i, t1)
                put(base + _BPW + i, t2)
                put(base + 2 * _BPW + i, t3)
                return 0

            lax.fori_loop(0, _BPW, body, 0, unroll=False)

        cp0.wait()
        head_2op(0, 0, 1, 0, m1)

        def nf2_body(i, _):
            t1 = []
            t2 = []
            for g in range(8):
                c1, c2 = group(2, i, g)
                d1, d2 = group(3, i, g)
                e1, er = group(4, i, g)
                start = jnp.maximum(c1 - c2, d1 - d2)
                end = jnp.minimum(c1 + c2, d1 + d2)
                new_r = (end - start) * 0.5
                cen = (start + end) * 0.5
                euc = jnp.abs(cen - e1)
                t1.append(jnp.maximum(euc + new_r - er + m1, 0.0))
                t2.append(jnp.maximum(start - end, 0.0))
            put(9 * _BPW + i, t1)
            put(10 * _BPW + i, t2)
            return 0

        lax.fori_loop(0, _BPW, nf2_body, 0, unroll=False)
        cp2.wait()

        def rel_head(base, cslot, dslot, rrow0, sgn):
            def body(i, _):
                dtail = re_rows[rrow0 + i, pl.ds(_DIM - _L + 1, _L)]
                delta = jnp.abs(dtail[_L - 1])
                bias = m1 - delta if sgn > 0 else m1 + delta
                t1 = []
                t2 = []
                t3 = []
                for g in range(8):
                    c1, cr = group(cslot, i, g)
                    d1, dr = group(dslot, i, g)
                    r1 = re_rows[rrow0 + i, pl.ds(g * _L, _L)]
                    euc = jnp.abs(c1 + r1 - d1) if sgn > 0 else jnp.abs(c1 - r1 - d1)
                    if sgn > 0:
                        t = euc + cr - dr + bias
                    else:
                        t = euc - cr - dr + bias
                    t1.append(jnp.maximum(t, 0.0))
                    t2.append(jnp.maximum(m - cr, 0.0))
                    t3.append(jnp.maximum(m - dr, 0.0))
                put(base + i, t1)
                put(base + _BPW + i, t2)
                put(base + 2 * _BPW + i, t3)
                put(base + 3 * _BPW + i,
                    jnp.where(lanes == _L - 1, dtail * dtail, 0.0))
                return 0

            lax.fori_loop(0, _BPW, body, 0, unroll=False)

        rel_head(11 * _BPW, 5, 6, 0, 1)
        cp1.wait()
        rel_head(15 * _BPW, 7, 8, _BPW, -1)
        head_2op(3 * _BPW, 9, 10, 1, m1)
        head_2op(6 * _BPW, 11, 12, -1, -m1)

        def top_body(i, _):
            t1 = []
            t2 = []
            for g in range(8):
                d1, dr = group(13, i, g)
                t1.append(jnp.maximum(_INF - dr * 0.5, 0.0))
                t2.append(jnp.maximum(_INF + d1, 0.0))
            put(19 * _BPW + i, t1)
            put(20 * _BPW + i, t2)
            return 0

        lax.fori_loop(0, _BPW, top_body, 0, unroll=False)

        zero = jnp.zeros((_L,), jnp.float32)
        for r in range(_ROWS, _PROWS * 8):
            buf[r // 8, pl.ds((r % 8) * _L, _L)] = zero
        pltpu.sync_copy(buf, out_hbm.at[pl.ds(w * _PROWS, _PROWS)])

    return _sc_loss


def _finalize_body(p_ref, out_ref):
    x = p_ref[...]
    lane = lax.broadcasted_iota(jnp.int32, x.shape, 1) // _L
    total = jnp.float32(0.0)
    for g in range(8):
        s = jnp.sum(jnp.where(lane == g, x, 0.0), axis=1)
        total = total + jnp.sum(jnp.sqrt(s))
    out_ref[0, 0] = total * (1.0 / _B)


def _finalize(partials):
    return pl.pallas_call(
        _finalize_body,
        out_shape=jax.ShapeDtypeStruct((1, 1), jnp.float32),
        out_specs=pl.BlockSpec(memory_space=pltpu.SMEM),
    )(partials)


def kernel(class_emb, rel_emb, nf1, nf2, nf3, nf4, disjoint, neg, top):
    ce_cols = jnp.stack([
        nf1[:_B, 0], nf1[:_B, 1],
        nf2[:_B, 0], nf2[:_B, 1], nf2[:_B, 2],
        nf3[:_B, 0], nf3[:_B, 2],
        nf4[:_B, 1], nf4[:_B, 2],
        disjoint[:_B, 0], disjoint[:_B, 1],
        neg[:_B, 0], neg[:_B, 1],
        top[:_B],
    ])
    re_cols = jnp.stack([nf3[:_B, 1], nf4[:_B, 0]])
    ce_idx = (ce_cols.reshape(_NCE, _NW, _BPW).transpose(1, 0, 2)
              .reshape(_NW, 2, 112))
    re_idx = (re_cols.reshape(_NRE, _NW, _BPW).transpose(1, 0, 2)
              .reshape(_NW, _NRE * _BPW))
    re_pad = jnp.pad(rel_emb, ((0, 0), (0, 2 * _DIM - (_DIM + 1))))
    partials = _get_sc_kernel()(class_emb, re_pad, ce_idx, re_idx)
    return _finalize(partials)[0, 0]


# speedup vs baseline: 1.0640x; 1.0640x over previous
import functools

import jax
import jax.numpy as jnp
from jax import lax
from jax.experimental import pallas as pl
from jax.experimental.pallas import tpu as pltpu
from jax.experimental.pallas import tpu_sc as plsc

_DIM = 128
_B = 512
_MARGIN = 0.1
_MARGIN1 = 0.05
_INF = 4.0
_NW = 32
_BPW = _B // _NW
_NCE = 14
_NRE = 2
_ROWS = 21 * _BPW
_PROWS = 48
_L = 16


@functools.cache
def _get_sc_kernel():
    mesh = plsc.VectorSubcoreMesh(core_axis_name="c", subcore_axis_name="s")

    @functools.partial(
        pl.kernel,
        mesh=mesh,
        out_type=jax.ShapeDtypeStruct((_NW * _PROWS, 128), jnp.float32),
        scratch_types=[
            pltpu.VMEM((2, 112), jnp.int32),
            pltpu.VMEM((_NRE * _BPW,), jnp.int32),
            pltpu.VMEM((_NCE * _BPW, 2 * _DIM), jnp.float32),
            pltpu.VMEM((_NRE * _BPW, 2 * _DIM), jnp.float32),
            pltpu.VMEM((_PROWS, 128), jnp.float32),
            pltpu.SemaphoreType.DMA,
        ],
    )
    def _sc_loss(ce_hbm, re_hbm, ceidx_hbm, reidx_hbm, out_hbm,
                 ceidx_v, reidx_v, ce_rows, re_rows, buf, sem):
        w = lax.axis_index("s") * 2 + lax.axis_index("c")
        pltpu.sync_copy(ceidx_hbm.at[w], ceidx_v)
        cp0 = pltpu.async_copy(ce_hbm.at[ceidx_v.at[0]], ce_rows.at[pl.ds(0, 112)], sem)
        cp1 = pltpu.async_copy(ce_hbm.at[ceidx_v.at[1]], ce_rows.at[pl.ds(112, 112)], sem)
        pltpu.sync_copy(reidx_hbm.at[w], reidx_v)
        cp2 = pltpu.async_copy(re_hbm.at[reidx_v], re_rows, sem)

        m = jnp.float32(_MARGIN)
        m1 = jnp.float32(_MARGIN1)
        lanes = lax.iota(jnp.int32, _L)

        def group(slot, i, g):
            a = ce_rows[slot * _BPW + i, pl.ds(g * _L, _L)]
            b = jnp.abs(ce_rows[slot * _BPW + i, pl.ds(_DIM + g * _L, _L)])
            return a, b

        def put(row, acc_or_parts):
            if isinstance(acc_or_parts, list):
                acc = acc_or_parts[0] * acc_or_parts[0]
                for p in acc_or_parts[1:]:
                    acc = acc + p * p
            else:
                acc = acc_or_parts
            buf[row // 8, pl.ds((row % 8) * _L, _L)] = acc

        def head_2op(base, cslot, dslot, sgn_r, bias):
            def body(i, _):
                t1 = []
                t2 = []
                t3 = []
                for g in range(8):
                    c1, cr = group(cslot, i, g)
                    d1, dr = group(dslot, i, g)
                    euc = jnp.abs(c1 - d1)
                    if sgn_r == 0:
                        t = euc + cr - dr + bias
                    elif sgn_r == 1:
                        t = cr + dr + bias - euc
                    else:
                        t = euc - cr - dr + bias
                    t1.append(jnp.maximum(t, 0.0))
                    t2.append(jnp.maximum(m - cr, 0.0))
                    t3.append(jnp.maximum(m - dr, 0.0))
                put(base + i, t1)
                put(base + _BPW + i, t2)
                put(base + 2 * _BPW + i, t3)
                return 0

            lax.fori_loop(0, _BPW, body, 0, unroll=False)

        cp0.wait()
        head_2op(0, 0, 1, 0, m1)

        def nf2_body(i, _):
            t1 = []
            t2 = []
            for g in range(8):
                c1, c2 = group(2, i, g)
                d1, d2 = group(3, i, g)
                e1, er = group(4, i, g)
                start = jnp.maximum(c1 - c2, d1 - d2)
                end = jnp.minimum(c1 + c2, d1 + d2)
                new_r = (end - start) * 0.5
                cen = (start + end) * 0.5
                euc = jnp.abs(cen - e1)
                t1.append(jnp.maximum(euc + new_r - er + m1, 0.0))
                t2.append(jnp.maximum(start - end, 0.0))
            put(9 * _BPW + i, t1)
            put(10 * _BPW + i, t2)
            return 0

        lax.fori_loop(0, _BPW, nf2_body, 0, unroll=False)
        cp2.wait()

        def rel_head(base, cslot, dslot, rrow0, sgn):
            def body(i, _):
                dtail = re_rows[rrow0 + i, pl.ds(_DIM - _L + 1, _L)]
                delta = jnp.abs(dtail[_L - 1])
                bias = m1 - delta if sgn > 0 else m1 + delta
                t1 = []
                t2 = []
                t3 = []
                for g in range(8):
                    c1, cr = group(cslot, i, g)
                    d1, dr = group(dslot, i, g)
                    r1 = re_rows[rrow0 + i, pl.ds(g * _L, _L)]
                    euc = jnp.abs(c1 + r1 - d1) if sgn > 0 else jnp.abs(c1 - r1 - d1)
                    if sgn > 0:
                        t = euc + cr - dr + bias
                    else:
                        t = euc - cr - dr + bias
                    t1.append(jnp.maximum(t, 0.0))
                    t2.append(jnp.maximum(m - cr, 0.0))
                    t3.append(jnp.maximum(m - dr, 0.0))
                put(base + i, t1)
                put(base + _BPW + i, t2)
                put(base + 2 * _BPW + i, t3)
                put(base + 3 * _BPW + i,
                    jnp.where(lanes == _L - 1, dtail * dtail, 0.0))
                return 0

            lax.fori_loop(0, _BPW, body, 0, unroll=False)

        rel_head(11 * _BPW, 5, 6, 0, 1)
        cp1.wait()
        rel_head(15 * _BPW, 7, 8, _BPW, -1)
        head_2op(3 * _BPW, 9, 10, 1, m1)
        head_2op(6 * _BPW, 11, 12, -1, -m1)

        def top_body(i, _):
            t1 = []
            t2 = []
            for g in range(8):
                d1, dr = group(13, i, g)
                t1.append(jnp.maximum(_INF - dr * 0.5, 0.0))
                t2.append(jnp.maximum(_INF + d1, 0.0))
            put(19 * _BPW + i, t1)
            put(20 * _BPW + i, t2)
            return 0

        lax.fori_loop(0, _BPW, top_body, 0, unroll=False)

        zero = jnp.zeros((_L,), jnp.float32)
        for r in range(_ROWS, _PROWS * 8):
            buf[r // 8, pl.ds((r % 8) * _L, _L)] = zero
        pltpu.sync_copy(buf, out_hbm.at[pl.ds(w * _PROWS, _PROWS)])

    return _sc_loss


def _finalize_body(p_ref, out_ref):
    x = p_ref[...]
    col = lax.broadcasted_iota(jnp.int32, (128, 8), 0) // _L
    grp = lax.broadcasted_iota(jnp.int32, (128, 8), 1)
    sel = (col == grp).astype(jnp.float32)
    x_hi = x.astype(jnp.bfloat16).astype(jnp.float32)
    x_lo = x - x_hi
    dn = (((1,), (0,)), ((), ()))
    sums = (lax.dot_general(x_hi, sel, dn, preferred_element_type=jnp.float32)
            + lax.dot_general(x_lo, sel, dn, preferred_element_type=jnp.float32))
    out_ref[0, 0] = jnp.sum(jnp.sqrt(sums)) * (1.0 / _B)


def _finalize(partials):
    return pl.pallas_call(
        _finalize_body,
        out_shape=jax.ShapeDtypeStruct((1, 1), jnp.float32),
        out_specs=pl.BlockSpec(memory_space=pltpu.SMEM),
    )(partials)


def kernel(class_emb, rel_emb, nf1, nf2, nf3, nf4, disjoint, neg, top):
    ce_cols = jnp.stack([
        nf1[:_B, 0], nf1[:_B, 1],
        nf2[:_B, 0], nf2[:_B, 1], nf2[:_B, 2],
        nf3[:_B, 0], nf3[:_B, 2],
        nf4[:_B, 1], nf4[:_B, 2],
        disjoint[:_B, 0], disjoint[:_B, 1],
        neg[:_B, 0], neg[:_B, 1],
        top[:_B],
    ])
    re_cols = jnp.stack([nf3[:_B, 1], nf4[:_B, 0]])
    ce_idx = (ce_cols.reshape(_NCE, _NW, _BPW).transpose(1, 0, 2)
              .reshape(_NW, 2, 112))
    re_idx = (re_cols.reshape(_NRE, _NW, _BPW).transpose(1, 0, 2)
              .reshape(_NW, _NRE * _BPW))
    re_pad = jnp.pad(rel_emb, ((0, 0), (0, 2 * _DIM - (_DIM + 1))))
    partials = _get_sc_kernel()(class_emb, re_pad, ce_idx, re_idx)
    return _finalize(partials)[0, 0]
